# baseline (speedup 1.0000x reference)
import functools

import jax
import jax.numpy as jnp
from jax import lax
from jax.experimental import pallas as pl
from jax.experimental.pallas import tpu as pltpu
from jax.experimental.pallas import tpu_sc as plsc

N = 10000
E = 160000
D_IN = 256
D_H = 256
D_OUT = 128
W128 = 128

NPAD = 10240
NC, NS, LANES = 2, 16, 16
NW = NC * NS
HALF = NPAD // NC
TRASH = HALF
HROWS = HALF + 8
EPT = E // NS
GB = 128
CAP = 11264
NG = CAP // GB
SB = 256
NG2 = CAP // SB
RPT = HALF // NS
BLK = 256


def _mesh():
    return plsc.VectorSubcoreMesh(
        core_axis_name="c", subcore_axis_name="s",
        num_cores=NC, num_subcores=NS)


def _route_body(src_hbm, dst_hbm,
                deg_hbm, sl_hbm, dl_hbm, cnt_hbm,
                sbuf, dbuf, slist, dlist, dlist2, ones_b, zbuf, cntv,
                degacc):
    c = lax.axis_index("c")
    s = lax.axis_index("s")
    wid = s * NC + c
    lo = c * HALF

    pltpu.sync_copy(src_hbm.at[pl.ds(s * EPT, EPT)], sbuf)
    pltpu.sync_copy(dst_hbm.at[pl.ds(s * EPT, EPT)], dbuf)

    zrow = jnp.zeros((LANES,), jnp.float32)
    for i in range(LANES):
        for jj in range(W128 // LANES):
            zbuf[i, pl.ds(jj * LANES, LANES)] = zrow
    for j in range(RPT // LANES):
        pltpu.sync_copy(zbuf, degacc.at[pl.ds(s * RPT + j * LANES, LANES)])

    e1 = (lax.iota(jnp.int32, LANES) == 0).astype(jnp.float32)

    def fill_ones(i, _):
        ones_b[i, pl.ds(0, LANES)] = e1
        for jj in range(1, W128 // LANES):
            ones_b[i, pl.ds(jj * LANES, LANES)] = zrow
        return 0
    lax.fori_loop(0, GB, fill_ones, 0, unroll=False)

    plsc.subcore_barrier()

    def filt(i, cnt):
        d = dbuf[pl.ds(i * LANES, LANES)]
        sv = sbuf[pl.ds(i * LANES, LANES)]
        m = (d >= lo) & (d < lo + HALF)
        plsc.store_compressed(slist.at[pl.ds(cnt, LANES)], sv, mask=m)
        plsc.store_compressed(dlist.at[pl.ds(cnt, LANES)], d - lo, mask=m)
        return cnt + jnp.sum(m.astype(jnp.int32))
    cnt = lax.fori_loop(0, EPT // LANES, filt, 0, unroll=False)

    zero16 = jnp.zeros((LANES,), jnp.int32)
    trash16 = jnp.full((LANES,), TRASH, jnp.int32)
    def pad_body(t, _):
        slist[pl.ds(cnt + t * LANES, LANES)] = zero16
        dlist[pl.ds(cnt + t * LANES, LANES)] = trash16
        return 0
    lax.fori_loop(0, 9 * GB // LANES, pad_body, 0, unroll=False)
    kpad = ((cnt + GB - 1) // GB) * GB
    nb = kpad // GB

    def relayout(r2, _):
        for h in range(2):
            for k in range(GB // LANES):
                dlist2[r2, h, pl.ds(k * LANES, LANES)] = dlist[
                    pl.ds((2 * r2 + h) * GB + k * LANES, LANES)]
        return 0
    lax.fori_loop(0, NG2, relayout, 0, unroll=False)

    nsb = (nb + 1) // 2

    def dscat(q, _):
        pltpu.sync_copy(ones_b, degacc.at[dlist2.at[q].at[0]], add=True)
        pltpu.sync_copy(ones_b, degacc.at[dlist2.at[q].at[1]], add=True)
        return 0
    lax.fori_loop(0, nsb, dscat, 0, unroll=False)

    plsc.subcore_barrier()

    pltpu.sync_copy(degacc.at[pl.ds(s * RPT, RPT)],
                    deg_hbm.at[pl.ds(c * HALF + s * RPT, RPT)])
    pltpu.sync_copy(slist, sl_hbm.at[wid])
    pltpu.sync_copy(dlist2, dl_hbm.at[wid])
    cntv[:] = jnp.full((LANES,), kpad, jnp.int32)
    pltpu.sync_copy(cntv, cnt_hbm.at[wid])


_route = pl.kernel(
    _route_body,
    out_type=[
        jax.ShapeDtypeStruct((NPAD, W128), jnp.float32),
        jax.ShapeDtypeStruct((NW, CAP), jnp.int32),
        jax.ShapeDtypeStruct((NW, NG2, 2, GB), jnp.int32),
        jax.ShapeDtypeStruct((NW, LANES), jnp.int32),
    ],
    mesh=_mesh(),
    compiler_params=pltpu.CompilerParams(needs_layout_passes=False),
    scratch_types=[
        pltpu.VMEM((EPT,), jnp.int32),
        pltpu.VMEM((EPT,), jnp.int32),
        pltpu.VMEM((CAP,), jnp.int32),
        pltpu.VMEM((CAP,), jnp.int32),
        pltpu.VMEM((NG2, 2, GB), jnp.int32),
        pltpu.VMEM((GB, W128), jnp.float32),
        pltpu.VMEM((LANES, W128), jnp.float32),
        pltpu.VMEM((LANES,), jnp.int32),
        pltpu.VMEM_SHARED((HROWS, W128), jnp.float32),
    ],
)


def _agg_body(nphases, *refs):
    hps = refs[:nphases]
    sl_hbm, dl_hbm, cnt_hbm = refs[nphases:nphases + 3]
    aggs = refs[nphases + 3:2 * nphases + 3]
    (slist, dbuf, cntv, zbuf, stag0, stag1,
     sem_g0, sem_g1, sem_s0, acc) = refs[2 * nphases + 3:]

    c = lax.axis_index("c")
    s = lax.axis_index("s")
    wid = s * NC + c

    pltpu.sync_copy(cnt_hbm.at[wid], cntv)
    pltpu.sync_copy(sl_hbm.at[wid], slist)

    zrow = jnp.zeros((LANES,), jnp.float32)

    def zfill(i, _):
        for jj in range(W128 // LANES):
            zbuf[i, pl.ds(jj * LANES, LANES)] = zrow
        return 0
    lax.fori_loop(0, 64, zfill, 0, unroll=False)

    nb = cntv[:][0] // GB
    nq4 = (nb + 3) // 4
    dl_w = dl_hbm.at[wid]

    def gidx(q):
        return slist.at[pl.ds(pl.multiple_of(q * SB, SB), SB)]

    def dload(q, k):
        pltpu.sync_copy(dl_w.at[pl.ds(q, 1)], dbuf.at[pl.ds(k, 1)])

    def scat2(stag, k):
        pltpu.async_copy(stag.at[pl.ds(0, GB)], acc.at[dbuf.at[k].at[0]],
                         sem_s0, add=True)
        pltpu.sync_copy(stag.at[pl.ds(GB, GB)], acc.at[dbuf.at[k].at[1]],
                        add=True)
        pltpu.make_async_copy(stag.at[pl.ds(0, GB)],
                              acc.at[dbuf.at[k].at[0]], sem_s0).wait()

    for p in range(nphases):
        for z in range(RPT // 64):
            pltpu.sync_copy(zbuf, acc.at[pl.ds(s * RPT + z * 64, 64)])
        plsc.subcore_barrier()

        dload(0, 0)
        pltpu.async_copy(hps[p].at[gidx(0)], stag0, sem_g0)

        def pair(u, _):
            q0 = 2 * u
            dload(q0 + 1, 1)
            pltpu.async_copy(hps[p].at[gidx(q0 + 1)], stag1, sem_g1)
            pltpu.make_async_copy(hps[p].at[gidx(q0)], stag0, sem_g0).wait()
            scat2(stag0, 0)
            dload(q0 + 2, 0)
            pltpu.async_copy(hps[p].at[gidx(q0 + 2)], stag0, sem_g0)
            pltpu.make_async_copy(hps[p].at[gidx(q0 + 1)], stag1,
                                  sem_g1).wait()
            scat2(stag1, 1)
            return 0
        lax.fori_loop(0, nq4, pair, 0, unroll=False)

        pltpu.make_async_copy(hps[p].at[gidx(0)], stag0, sem_g0).wait()
        plsc.subcore_barrier()
        pltpu.sync_copy(acc.at[pl.ds(s * RPT, RPT)],
                        aggs[p].at[pl.ds(c * HALF + s * RPT, RPT)])
        plsc.subcore_barrier()


def _make_agg(nphases):
    return pl.kernel(
        functools.partial(_agg_body, nphases),
        out_type=[jax.ShapeDtypeStruct((NPAD, W128), jnp.float32)
                  for _ in range(nphases)],
        mesh=_mesh(),
        compiler_params=pltpu.CompilerParams(needs_layout_passes=False),
        scratch_types=[
            pltpu.VMEM((CAP,), jnp.int32),
            pltpu.VMEM((2, 2, GB), jnp.int32),
            pltpu.VMEM((LANES,), jnp.int32),
            pltpu.VMEM((64, W128), jnp.float32),
            pltpu.VMEM((SB, W128), jnp.float32),
            pltpu.VMEM((SB, W128), jnp.float32),
            pltpu.SemaphoreType.DMA,
            pltpu.SemaphoreType.DMA,
            pltpu.SemaphoreType.DMA,
            pltpu.VMEM_SHARED((HROWS, W128), jnp.float32),
        ],
    )


_agg2 = _make_agg(2)
_agg1 = _make_agg(1)


def _tc0_body(deg_ref, x_ref, w_ref, dis_ref, hp0_ref, hp1_ref):
    dis = lax.rsqrt(deg_ref[:, 0:1] + 1.0)
    dis_ref[...] = jnp.broadcast_to(dis, (BLK, 128))
    h = dis * jnp.dot(x_ref[...], w_ref[...],
                      preferred_element_type=jnp.float32)
    hp0_ref[...] = h[:, :W128]
    hp1_ref[...] = h[:, W128:]


_tc0 = pl.pallas_call(
    _tc0_body,
    grid=(NPAD // BLK,),
    in_specs=[
        pl.BlockSpec((BLK, W128), lambda i: (i, 0)),
        pl.BlockSpec((BLK, D_IN), lambda i: (i, 0)),
        pl.BlockSpec((D_IN, D_H), lambda i: (0, 0)),
    ],
    out_specs=[
        pl.BlockSpec((BLK, 128), lambda i: (i, 0)),
        pl.BlockSpec((BLK, W128), lambda i: (i, 0)),
        pl.BlockSpec((BLK, W128), lambda i: (i, 0)),
    ],
    out_shape=[
        jax.ShapeDtypeStruct((NPAD, 128), jnp.float32),
        jax.ShapeDtypeStruct((NPAD, W128), jnp.float32),
        jax.ShapeDtypeStruct((NPAD, W128), jnp.float32),
    ],
)


def _tcmid_body(split_out, a0_ref, a1_ref, h0_ref, h1_ref, dis_ref, b_ref,
                w_ref, *o_refs):
    dis = dis_ref[:, 0:1]
    agg = jnp.concatenate([a0_ref[...], a1_ref[...]], axis=1)
    hp = jnp.concatenate([h0_ref[...], h1_ref[...]], axis=1)
    t = jnp.maximum(dis * (agg + hp) + b_ref[...], 0.0)
    out = dis * jnp.dot(t, w_ref[...], preferred_element_type=jnp.float32)
    if split_out:
        o_refs[0][...] = out[:, :W128]
        o_refs[1][...] = out[:, W128:]
    else:
        o_refs[0][...] = out


def _make_tcmid(d_out, split_out):
    n_out = 2 if split_out else 1
    return pl.pallas_call(
        functools.partial(_tcmid_body, split_out),
        grid=(NPAD // BLK,),
        in_specs=[
            pl.BlockSpec((BLK, W128), lambda i: (i, 0)),
            pl.BlockSpec((BLK, W128), lambda i: (i, 0)),
            pl.BlockSpec((BLK, W128), lambda i: (i, 0)),
            pl.BlockSpec((BLK, W128), lambda i: (i, 0)),
            pl.BlockSpec((BLK, 128), lambda i: (i, 0)),
            pl.BlockSpec((1, D_H), lambda i: (0, 0)),
            pl.BlockSpec((D_H, d_out), lambda i: (0, 0)),
        ],
        out_specs=[pl.BlockSpec((BLK, d_out // n_out), lambda i: (i, 0))
                   for _ in range(n_out)],
        out_shape=[jax.ShapeDtypeStruct((NPAD, d_out // n_out), jnp.float32)
                   for _ in range(n_out)],
    )


_tcmid_a = _make_tcmid(D_H, True)
_tcmid_b = _make_tcmid(D_OUT, False)


def _tcfin_body(agg_ref, hp_ref, dis_ref, b_ref, o_ref):
    dis = dis_ref[:, 0:1]
    u = dis * (agg_ref[...] + hp_ref[...]) + b_ref[...]
    m = jnp.max(u, axis=1, keepdims=True)
    lse = m + jnp.log(jnp.sum(jnp.exp(u - m), axis=1, keepdims=True))
    o_ref[...] = u - lse


_tcfin = pl.pallas_call(
    _tcfin_body,
    grid=(NPAD // BLK,),
    in_specs=[
        pl.BlockSpec((BLK, D_OUT), lambda i: (i, 0)),
        pl.BlockSpec((BLK, D_OUT), lambda i: (i, 0)),
        pl.BlockSpec((BLK, 128), lambda i: (i, 0)),
        pl.BlockSpec((1, D_OUT), lambda i: (0, 0)),
    ],
    out_specs=pl.BlockSpec((BLK, D_OUT), lambda i: (i, 0)),
    out_shape=jax.ShapeDtypeStruct((NPAD, D_OUT), jnp.float32),
)


def kernel(x, edge_index, W1, b1, W2, b2, W3, b3):
    ei = edge_index.astype(jnp.int32)
    src, dst = ei[0], ei[1]
    xp = jnp.zeros((NPAD, D_IN), jnp.float32).at[:N].set(x)

    deg, slists, dlists, cnts = _route(src, dst)
    dis, hp1a, hp1b = _tc0(deg, xp, W1)
    agg1a, agg1b = _agg2(hp1a, hp1b, slists, dlists, cnts)
    hp2a, hp2b = _tcmid_a(agg1a, agg1b, hp1a, hp1b, dis,
                          b1.reshape(1, -1), W2)
    agg2a, agg2b = _agg2(hp2a, hp2b, slists, dlists, cnts)
    hp3 = _tcmid_b(agg2a, agg2b, hp2a, hp2b, dis, b2.reshape(1, -1), W3)[0]
    agg3 = _agg1(hp3, slists, dlists, cnts)[0]
    out = _tcfin(agg3, hp3, dis, b3.reshape(1, -1))
    return out[:N]


# speedup vs baseline: 1.5966x; 1.5966x over previous
import functools

import jax
import jax.numpy as jnp
from jax import lax
from jax.experimental import pallas as pl
from jax.experimental.pallas import tpu as pltpu
from jax.experimental.pallas import tpu_sc as plsc

N = 10000
E = 160000
D_IN = 256
D_H = 256
D_OUT = 128
W128 = 128

NPAD = 10240
NC, NS, LANES = 2, 16, 16
NW = NC * NS
HALF = NPAD // NC
TRASH = HALF
HROWS = HALF + 8
EPT = E // NS
GB = 128
CAP = 11264
NG = CAP // GB
SB = 256
NG2 = CAP // SB
RPT = HALF // NS
BLK = 256


def _mesh():
    return plsc.VectorSubcoreMesh(
        core_axis_name="c", subcore_axis_name="s",
        num_cores=NC, num_subcores=NS)


def _route_body(src_hbm, dst_hbm,
                deg_hbm, sl_hbm, dl_hbm, cnt_hbm,
                sbuf, dbuf, slist, dlist, slist2, dlist2, ones_b, zbuf,
                cntv, degacc):
    c = lax.axis_index("c")
    s = lax.axis_index("s")
    wid = s * NC + c
    lo = c * HALF

    pltpu.sync_copy(src_hbm.at[pl.ds(s * EPT, EPT)], sbuf)
    pltpu.sync_copy(dst_hbm.at[pl.ds(s * EPT, EPT)], dbuf)

    zrow = jnp.zeros((LANES,), jnp.float32)
    for i in range(LANES):
        for jj in range(W128 // LANES):
            zbuf[i, pl.ds(jj * LANES, LANES)] = zrow
    for j in range(RPT // LANES):
        pltpu.sync_copy(zbuf, degacc.at[pl.ds(s * RPT + j * LANES, LANES)])

    e1 = (lax.iota(jnp.int32, LANES) == 0).astype(jnp.float32)

    def fill_ones(i, _):
        ones_b[i, pl.ds(0, LANES)] = e1
        for jj in range(1, W128 // LANES):
            ones_b[i, pl.ds(jj * LANES, LANES)] = zrow
        return 0
    lax.fori_loop(0, GB, fill_ones, 0, unroll=False)

    plsc.subcore_barrier()

    def filt(i, cnt):
        d = dbuf[pl.ds(i * LANES, LANES)]
        sv = sbuf[pl.ds(i * LANES, LANES)]
        m = (d >= lo) & (d < lo + HALF)
        plsc.store_compressed(slist.at[pl.ds(cnt, LANES)], sv, mask=m)
        plsc.store_compressed(dlist.at[pl.ds(cnt, LANES)], d - lo, mask=m)
        return cnt + jnp.sum(m.astype(jnp.int32))
    cnt = lax.fori_loop(0, EPT // LANES, filt, 0, unroll=False)

    zero16 = jnp.zeros((LANES,), jnp.int32)
    trash16 = jnp.full((LANES,), TRASH, jnp.int32)
    def pad_body(t, _):
        slist[pl.ds(cnt + t * LANES, LANES)] = zero16
        dlist[pl.ds(cnt + t * LANES, LANES)] = trash16
        return 0
    lax.fori_loop(0, 9 * GB // LANES, pad_body, 0, unroll=False)
    kpad = ((cnt + GB - 1) // GB) * GB
    nb = kpad // GB

    def relayout(r, _):
        for k in range(GB // LANES):
            slist2[r, pl.ds(k * LANES, LANES)] = slist[
                pl.ds(r * GB + k * LANES, LANES)]
            dlist2[r, pl.ds(k * LANES, LANES)] = dlist[
                pl.ds(r * GB + k * LANES, LANES)]
        return 0
    lax.fori_loop(0, NG, relayout, 0, unroll=False)

    def dscat(j, _):
        pltpu.sync_copy(ones_b, degacc.at[dlist2.at[j]], add=True)
        return 0
    lax.fori_loop(0, nb, dscat, 0, unroll=False)

    plsc.subcore_barrier()

    pltpu.sync_copy(degacc.at[pl.ds(s * RPT, RPT)],
                    deg_hbm.at[pl.ds(c * HALF + s * RPT, RPT)])
    pltpu.sync_copy(slist2, sl_hbm.at[wid])
    pltpu.sync_copy(dlist2, dl_hbm.at[wid])
    cntv[:] = jnp.full((LANES,), kpad, jnp.int32)
    pltpu.sync_copy(cntv, cnt_hbm.at[wid])


_route = pl.kernel(
    _route_body,
    out_type=[
        jax.ShapeDtypeStruct((NPAD, W128), jnp.float32),
        jax.ShapeDtypeStruct((NW, NG, GB), jnp.int32),
        jax.ShapeDtypeStruct((NW, NG, GB), jnp.int32),
        jax.ShapeDtypeStruct((NW, LANES), jnp.int32),
    ],
    mesh=_mesh(),
    compiler_params=pltpu.CompilerParams(needs_layout_passes=False),
    scratch_types=[
        pltpu.VMEM((EPT,), jnp.int32),
        pltpu.VMEM((EPT,), jnp.int32),
        pltpu.VMEM((CAP,), jnp.int32),
        pltpu.VMEM((CAP,), jnp.int32),
        pltpu.VMEM((NG, GB), jnp.int32),
        pltpu.VMEM((NG, GB), jnp.int32),
        pltpu.VMEM((GB, W128), jnp.float32),
        pltpu.VMEM((LANES, W128), jnp.float32),
        pltpu.VMEM((LANES,), jnp.int32),
        pltpu.VMEM_SHARED((HROWS, W128), jnp.float32),
    ],
)


def _agg_body(nphases, *refs):
    hps = refs[:nphases]
    sl_hbm, dl_hbm, cnt_hbm = refs[nphases:nphases + 3]
    aggs = refs[nphases + 3:2 * nphases + 3]
    (slist2, dlist2, cntv, zbuf, stag0, stag1,
     sem_g0, sem_g1, sem_s0, acc) = refs[2 * nphases + 3:]

    c = lax.axis_index("c")
    s = lax.axis_index("s")
    wid = s * NC + c

    pltpu.sync_copy(cnt_hbm.at[wid], cntv)
    pltpu.sync_copy(sl_hbm.at[wid], slist2)
    pltpu.sync_copy(dl_hbm.at[wid], dlist2)

    zrow = jnp.zeros((LANES,), jnp.float32)

    def zfill(i, _):
        for jj in range(W128 // LANES):
            zbuf[i, pl.ds(jj * LANES, LANES)] = zrow
        return 0
    lax.fori_loop(0, 64, zfill, 0, unroll=False)

    nb = cntv[:][0] // GB
    npair = (nb + 1) // 2

    for p in range(nphases):
        for z in range(RPT // 64):
            pltpu.sync_copy(zbuf, acc.at[pl.ds(s * RPT + z * 64, 64)])
        plsc.subcore_barrier()

        pltpu.async_copy(hps[p].at[slist2.at[0]], stag0, sem_g0)

        def pair(u, _):
            j0 = 2 * u
            pltpu.async_copy(hps[p].at[slist2.at[j0 + 1]], stag1, sem_g1)
            pltpu.make_async_copy(hps[p].at[slist2.at[j0]], stag0,
                                  sem_g0).wait()
            pltpu.async_copy(stag0, acc.at[dlist2.at[j0]], sem_s0,
                             add=True)
            pltpu.make_async_copy(hps[p].at[slist2.at[j0 + 1]], stag1,
                                  sem_g1).wait()
            pltpu.sync_copy(stag1, acc.at[dlist2.at[j0 + 1]], add=True)
            pltpu.make_async_copy(stag0, acc.at[dlist2.at[j0]],
                                  sem_s0).wait()
            pltpu.async_copy(hps[p].at[slist2.at[j0 + 2]], stag0, sem_g0)
            return 0
        lax.fori_loop(0, npair, pair, 0, unroll=False)

        pltpu.make_async_copy(hps[p].at[slist2.at[0]], stag0, sem_g0).wait()
        plsc.subcore_barrier()
        pltpu.sync_copy(acc.at[pl.ds(s * RPT, RPT)],
                        aggs[p].at[pl.ds(c * HALF + s * RPT, RPT)])
        plsc.subcore_barrier()


def _make_agg(nphases):
    return pl.kernel(
        functools.partial(_agg_body, nphases),
        out_type=[jax.ShapeDtypeStruct((NPAD, W128), jnp.float32)
                  for _ in range(nphases)],
        mesh=_mesh(),
        compiler_params=pltpu.CompilerParams(needs_layout_passes=False),
        scratch_types=[
            pltpu.VMEM((NG, GB), jnp.int32),
            pltpu.VMEM((NG, GB), jnp.int32),
            pltpu.VMEM((LANES,), jnp.int32),
            pltpu.VMEM((64, W128), jnp.float32),
            pltpu.VMEM((GB, W128), jnp.float32),
            pltpu.VMEM((GB, W128), jnp.float32),
            pltpu.SemaphoreType.DMA,
            pltpu.SemaphoreType.DMA,
            pltpu.SemaphoreType.DMA,
            pltpu.VMEM_SHARED((HROWS, W128), jnp.float32),
        ],
    )


_agg2 = _make_agg(2)
_agg1 = _make_agg(1)


def _tc0_body(deg_ref, x_ref, w_ref, dis_ref, hp0_ref, hp1_ref):
    dis = lax.rsqrt(deg_ref[:, 0:1] + 1.0)
    dis_ref[...] = jnp.broadcast_to(dis, (BLK, 128))
    h = dis * jnp.dot(x_ref[...], w_ref[...],
                      preferred_element_type=jnp.float32)
    hp0_ref[...] = h[:, :W128]
    hp1_ref[...] = h[:, W128:]


_tc0 = pl.pallas_call(
    _tc0_body,
    grid=(NPAD // BLK,),
    in_specs=[
        pl.BlockSpec((BLK, W128), lambda i: (i, 0)),
        pl.BlockSpec((BLK, D_IN), lambda i: (i, 0)),
        pl.BlockSpec((D_IN, D_H), lambda i: (0, 0)),
    ],
    out_specs=[
        pl.BlockSpec((BLK, 128), lambda i: (i, 0)),
        pl.BlockSpec((BLK, W128), lambda i: (i, 0)),
        pl.BlockSpec((BLK, W128), lambda i: (i, 0)),
    ],
    out_shape=[
        jax.ShapeDtypeStruct((NPAD, 128), jnp.float32),
        jax.ShapeDtypeStruct((NPAD, W128), jnp.float32),
        jax.ShapeDtypeStruct((NPAD, W128), jnp.float32),
    ],
)


def _tcmid_body(split_out, a0_ref, a1_ref, h0_ref, h1_ref, dis_ref, b_ref,
                w_ref, *o_refs):
    dis = dis_ref[:, 0:1]
    agg = jnp.concatenate([a0_ref[...], a1_ref[...]], axis=1)
    hp = jnp.concatenate([h0_ref[...], h1_ref[...]], axis=1)
    t = jnp.maximum(dis * (agg + hp) + b_ref[...], 0.0)
    out = dis * jnp.dot(t, w_ref[...], preferred_element_type=jnp.float32)
    if split_out:
        o_refs[0][...] = out[:, :W128]
        o_refs[1][...] = out[:, W128:]
    else:
        o_refs[0][...] = out


def _make_tcmid(d_out, split_out):
    n_out = 2 if split_out else 1
    return pl.pallas_call(
        functools.partial(_tcmid_body, split_out),
        grid=(NPAD // BLK,),
        in_specs=[
            pl.BlockSpec((BLK, W128), lambda i: (i, 0)),
            pl.BlockSpec((BLK, W128), lambda i: (i, 0)),
            pl.BlockSpec((BLK, W128), lambda i: (i, 0)),
            pl.BlockSpec((BLK, W128), lambda i: (i, 0)),
            pl.BlockSpec((BLK, 128), lambda i: (i, 0)),
            pl.BlockSpec((1, D_H), lambda i: (0, 0)),
            pl.BlockSpec((D_H, d_out), lambda i: (0, 0)),
        ],
        out_specs=[pl.BlockSpec((BLK, d_out // n_out), lambda i: (i, 0))
                   for _ in range(n_out)],
        out_shape=[jax.ShapeDtypeStruct((NPAD, d_out // n_out), jnp.float32)
                   for _ in range(n_out)],
    )


_tcmid_a = _make_tcmid(D_H, True)
_tcmid_b = _make_tcmid(D_OUT, False)


def _tcfin_body(agg_ref, hp_ref, dis_ref, b_ref, o_ref):
    dis = dis_ref[:, 0:1]
    u = dis * (agg_ref[...] + hp_ref[...]) + b_ref[...]
    m = jnp.max(u, axis=1, keepdims=True)
    lse = m + jnp.log(jnp.sum(jnp.exp(u - m), axis=1, keepdims=True))
    o_ref[...] = u - lse


_tcfin = pl.pallas_call(
    _tcfin_body,
    grid=(NPAD // BLK,),
    in_specs=[
        pl.BlockSpec((BLK, D_OUT), lambda i: (i, 0)),
        pl.BlockSpec((BLK, D_OUT), lambda i: (i, 0)),
        pl.BlockSpec((BLK, 128), lambda i: (i, 0)),
        pl.BlockSpec((1, D_OUT), lambda i: (0, 0)),
    ],
    out_specs=pl.BlockSpec((BLK, D_OUT), lambda i: (i, 0)),
    out_shape=jax.ShapeDtypeStruct((NPAD, D_OUT), jnp.float32),
)


def kernel(x, edge_index, W1, b1, W2, b2, W3, b3):
    ei = edge_index.astype(jnp.int32)
    src, dst = ei[0], ei[1]
    xp = jnp.zeros((NPAD, D_IN), jnp.float32).at[:N].set(x)

    deg, slists, dlists, cnts = _route(src, dst)
    dis, hp1a, hp1b = _tc0(deg, xp, W1)
    agg1a, agg1b = _agg2(hp1a, hp1b, slists, dlists, cnts)
    hp2a, hp2b = _tcmid_a(agg1a, agg1b, hp1a, hp1b, dis,
                          b1.reshape(1, -1), W2)
    agg2a, agg2b = _agg2(hp2a, hp2b, slists, dlists, cnts)
    hp3 = _tcmid_b(agg2a, agg2b, hp2a, hp2b, dis, b2.reshape(1, -1), W3)[0]
    agg3 = _agg1(hp3, slists, dlists, cnts)[0]
    out = _tcfin(agg3, hp3, dis, b3.reshape(1, -1))
    return out[:N]


# speedup vs baseline: 3.7745x; 2.3641x over previous
import functools

import jax
import jax.numpy as jnp
from jax import lax
from jax.experimental import pallas as pl
from jax.experimental.pallas import tpu as pltpu
from jax.experimental.pallas import tpu_sc as plsc

N = 10000
E = 160000
D_IN = 256
D_H = 256
D_OUT = 128
W128 = 128

NPAD = 10240
NC, NS, LANES = 2, 16, 16
NW = NC * NS
HALF = NPAD // NC
TRASH = HALF
HROWS = HALF + 8
EPT = E // NS
GB = 128
CAP = 10496
NG = CAP // GB
RPT = HALF // NS
BLK = 256


def _mesh():
    return plsc.VectorSubcoreMesh(
        core_axis_name="c", subcore_axis_name="s",
        num_cores=NC, num_subcores=NS)


def _route_body(src_hbm, dst_hbm,
                deg_hbm, sl_hbm, dl_hbm, cnt_hbm,
                sbuf, dbuf, slist, dlist, dlist2, ones_b, zbuf, cntv,
                degacc):
    c = lax.axis_index("c")
    s = lax.axis_index("s")
    wid = s * NC + c
    lo = c * HALF

    pltpu.sync_copy(src_hbm.at[pl.ds(s * EPT, EPT)], sbuf)
    pltpu.sync_copy(dst_hbm.at[pl.ds(s * EPT, EPT)], dbuf)

    zrow = jnp.zeros((LANES,), jnp.float32)
    for i in range(LANES):
        for jj in range(W128 // LANES):
            zbuf[i, pl.ds(jj * LANES, LANES)] = zrow
    for j in range(RPT // LANES):
        pltpu.sync_copy(zbuf, degacc.at[pl.ds(s * RPT + j * LANES, LANES)])

    e1 = (lax.iota(jnp.int32, LANES) == 0).astype(jnp.float32)

    def fill_ones(i, _):
        ones_b[i, pl.ds(0, LANES)] = e1
        for jj in range(1, W128 // LANES):
            ones_b[i, pl.ds(jj * LANES, LANES)] = zrow
        return 0
    lax.fori_loop(0, GB, fill_ones, 0, unroll=False)

    plsc.subcore_barrier()

    def filt(i, cnt):
        d = dbuf[pl.ds(i * LANES, LANES)]
        sv = sbuf[pl.ds(i * LANES, LANES)]
        m = (d >= lo) & (d < lo + HALF)
        plsc.store_compressed(slist.at[pl.ds(cnt, LANES)], sv, mask=m)
        plsc.store_compressed(dlist.at[pl.ds(cnt, LANES)], d - lo, mask=m)
        return cnt + jnp.sum(m.astype(jnp.int32))
    cnt = lax.fori_loop(0, EPT // LANES, filt, 0, unroll=False)

    zero16 = jnp.zeros((LANES,), jnp.int32)
    trash16 = jnp.full((LANES,), TRASH, jnp.int32)
    for t in range(2 * GB // LANES):
        slist[pl.ds(cnt + t * LANES, LANES)] = zero16
        dlist[pl.ds(cnt + t * LANES, LANES)] = trash16
    kpad = ((cnt + GB - 1) // GB) * GB
    nb = kpad // GB

    def relayout(r, _):
        for k in range(GB // LANES):
            dlist2[r, pl.ds(k * LANES, LANES)] = dlist[
                pl.ds(r * GB + k * LANES, LANES)]
        return 0
    lax.fori_loop(0, NG, relayout, 0, unroll=False)

    def dscat(j, _):
        pltpu.sync_copy(ones_b, degacc.at[dlist2.at[j]], add=True)
        return 0
    lax.fori_loop(0, nb, dscat, 0, unroll=False)

    plsc.subcore_barrier()

    pltpu.sync_copy(degacc.at[pl.ds(s * RPT, RPT)],
                    deg_hbm.at[pl.ds(c * HALF + s * RPT, RPT)])
    pltpu.sync_copy(slist, sl_hbm.at[wid])
    pltpu.sync_copy(dlist2, dl_hbm.at[wid])
    cntv[:] = jnp.full((LANES,), kpad, jnp.int32)
    pltpu.sync_copy(cntv, cnt_hbm.at[wid])


_route = pl.kernel(
    _route_body,
    out_type=[
        jax.ShapeDtypeStruct((NPAD, W128), jnp.float32),
        jax.ShapeDtypeStruct((NW, CAP), jnp.int32),
        jax.ShapeDtypeStruct((NW, NG, GB), jnp.int32),
        jax.ShapeDtypeStruct((NW, LANES), jnp.int32),
    ],
    mesh=_mesh(),
    compiler_params=pltpu.CompilerParams(needs_layout_passes=False),
    scratch_types=[
        pltpu.VMEM((EPT,), jnp.int32),
        pltpu.VMEM((EPT,), jnp.int32),
        pltpu.VMEM((CAP,), jnp.int32),
        pltpu.VMEM((CAP,), jnp.int32),
        pltpu.VMEM((NG, GB), jnp.int32),
        pltpu.VMEM((GB, W128), jnp.float32),
        pltpu.VMEM((LANES, W128), jnp.float32),
        pltpu.VMEM((LANES,), jnp.int32),
        pltpu.VMEM_SHARED((HROWS, W128), jnp.float32),
    ],
)


def _agg_body(nplanes, *refs):
    hps = refs[:nplanes]
    sl_hbm, dl_hbm, cnt_hbm = refs[nplanes:nplanes + 3]
    aggs = refs[nplanes + 3:2 * nplanes + 3]
    sbufl, dbufl, cntv, zbuf = refs[2 * nplanes + 3:2 * nplanes + 7]
    stag0, stag1 = refs[2 * nplanes + 7:2 * nplanes + 9]
    sem_g0, sem_g1, sem_s0 = refs[2 * nplanes + 9:2 * nplanes + 12]
    accs = refs[2 * nplanes + 12:]

    c = lax.axis_index("c")
    s = lax.axis_index("s")
    wid = s * NC + c

    pltpu.sync_copy(cnt_hbm.at[wid], cntv)

    zrow = jnp.zeros((LANES,), jnp.float32)
    for i in range(LANES):
        for jj in range(W128 // LANES):
            zbuf[i, pl.ds(jj * LANES, LANES)] = zrow
    for j in range(RPT // LANES):
        for p in range(nplanes):
            pltpu.sync_copy(zbuf,
                            accs[p].at[pl.ds(s * RPT + j * LANES, LANES)])

    plsc.subcore_barrier()

    nb = cntv[:][0] // GB

    sl_w = sl_hbm.at[wid]
    dl_w = dl_hbm.at[wid]

    def load_lists(j, slot):
        pltpu.sync_copy(sl_w.at[pl.ds(j * GB, GB)], sbufl.at[slot])
        pltpu.sync_copy(dl_w.at[pl.ds(j, 1)], dbufl.at[pl.ds(slot, 1)])

    if nplanes == 2:
        @pl.when(nb > 0)
        def _prologue():
            load_lists(0, 0)
            pltpu.async_copy(hps[0].at[sbufl.at[0]], stag0, sem_g0)

        def body(j, _):
            a = j % 2
            b = 1 - a
            pltpu.make_async_copy(hps[0].at[sbufl.at[a]], stag0,
                                  sem_g0).wait()
            pltpu.async_copy(hps[1].at[sbufl.at[a]], stag1, sem_g1)
            pltpu.async_copy(stag0, accs[0].at[dbufl.at[a]], sem_s0,
                             add=True)
            load_lists(j + 1, b)
            pltpu.make_async_copy(stag0, accs[0].at[dbufl.at[a]],
                                  sem_s0).wait()

            @pl.when(j + 1 < nb)
            def _next():
                pltpu.async_copy(hps[0].at[sbufl.at[b]], stag0, sem_g0)
            pltpu.make_async_copy(hps[1].at[sbufl.at[a]], stag1,
                                  sem_g1).wait()
            pltpu.sync_copy(stag1, accs[1].at[dbufl.at[a]], add=True)
            return 0
        lax.fori_loop(0, nb, body, 0, unroll=False)
    else:
        npairs = nb // 2
        rem = nb - 2 * npairs

        @pl.when(nb > 0)
        def _prologue():
            load_lists(0, 0)
            pltpu.async_copy(hps[0].at[sbufl.at[0]], stag0, sem_g0)

        def pbody(u, _):
            j0 = 2 * u
            load_lists(j0 + 1, 1)
            pltpu.async_copy(hps[0].at[sbufl.at[1]], stag1, sem_g1)
            pltpu.make_async_copy(hps[0].at[sbufl.at[0]], stag0,
                                  sem_g0).wait()
            pltpu.async_copy(stag0, accs[0].at[dbufl.at[0]], sem_s0,
                             add=True)
            pltpu.make_async_copy(hps[0].at[sbufl.at[1]], stag1,
                                  sem_g1).wait()
            pltpu.sync_copy(stag1, accs[0].at[dbufl.at[1]], add=True)
            pltpu.make_async_copy(stag0, accs[0].at[dbufl.at[0]],
                                  sem_s0).wait()
            load_lists(j0 + 2, 0)
            pltpu.async_copy(hps[0].at[sbufl.at[0]], stag0, sem_g0)
            return 0
        lax.fori_loop(0, npairs, pbody, 0, unroll=False)

        @pl.when(nb > 0)
        def _epilogue():
            pltpu.make_async_copy(hps[0].at[sbufl.at[0]], stag0,
                                  sem_g0).wait()

            @pl.when(rem == 1)
            def _tail():
                pltpu.sync_copy(stag0, accs[0].at[dbufl.at[0]], add=True)

    plsc.subcore_barrier()

    for p in range(nplanes):
        pltpu.sync_copy(accs[p].at[pl.ds(s * RPT, RPT)],
                        aggs[p].at[pl.ds(c * HALF + s * RPT, RPT)])


def _make_agg(nplanes):
    return pl.kernel(
        functools.partial(_agg_body, nplanes),
        out_type=[jax.ShapeDtypeStruct((NPAD, W128), jnp.float32)
                  for _ in range(nplanes)],
        mesh=_mesh(),
        compiler_params=pltpu.CompilerParams(needs_layout_passes=False),
        scratch_types=(
            [pltpu.VMEM((2, GB), jnp.int32),
             pltpu.VMEM((2, GB), jnp.int32),
             pltpu.VMEM((LANES,), jnp.int32),
             pltpu.VMEM((LANES, W128), jnp.float32),
             pltpu.VMEM((GB, W128), jnp.float32),
             pltpu.VMEM((GB, W128), jnp.float32),
             pltpu.SemaphoreType.DMA,
             pltpu.SemaphoreType.DMA,
             pltpu.SemaphoreType.DMA]
            + [pltpu.VMEM_SHARED((HROWS, W128), jnp.float32)
               for _ in range(nplanes)]
        ),
    )


_agg2 = _make_agg(2)
_agg1 = _make_agg(1)


def _tc0_body(deg_ref, x_ref, w_ref, dis_ref, hp0_ref, hp1_ref):
    dis = lax.rsqrt(deg_ref[:, 0:1] + 1.0)
    dis_ref[...] = jnp.broadcast_to(dis, (BLK, 128))
    h = dis * jnp.dot(x_ref[...], w_ref[...],
                      preferred_element_type=jnp.float32)
    hp0_ref[...] = h[:, :W128]
    hp1_ref[...] = h[:, W128:]


_tc0 = pl.pallas_call(
    _tc0_body,
    grid=(NPAD // BLK,),
    in_specs=[
        pl.BlockSpec((BLK, W128), lambda i: (i, 0)),
        pl.BlockSpec((BLK, D_IN), lambda i: (i, 0)),
        pl.BlockSpec((D_IN, D_H), lambda i: (0, 0)),
    ],
    out_specs=[
        pl.BlockSpec((BLK, 128), lambda i: (i, 0)),
        pl.BlockSpec((BLK, W128), lambda i: (i, 0)),
        pl.BlockSpec((BLK, W128), lambda i: (i, 0)),
    ],
    out_shape=[
        jax.ShapeDtypeStruct((NPAD, 128), jnp.float32),
        jax.ShapeDtypeStruct((NPAD, W128), jnp.float32),
        jax.ShapeDtypeStruct((NPAD, W128), jnp.float32),
    ],
)


def _tcmid_body(split_out, a0_ref, a1_ref, h0_ref, h1_ref, dis_ref, b_ref,
                w_ref, *o_refs):
    dis = dis_ref[:, 0:1]
    agg = jnp.concatenate([a0_ref[...], a1_ref[...]], axis=1)
    hp = jnp.concatenate([h0_ref[...], h1_ref[...]], axis=1)
    t = jnp.maximum(dis * (agg + hp) + b_ref[...], 0.0)
    out = dis * jnp.dot(t, w_ref[...], preferred_element_type=jnp.float32)
    if split_out:
        o_refs[0][...] = out[:, :W128]
        o_refs[1][...] = out[:, W128:]
    else:
        o_refs[0][...] = out


def _make_tcmid(d_out, split_out):
    n_out = 2 if split_out else 1
    return pl.pallas_call(
        functools.partial(_tcmid_body, split_out),
        grid=(NPAD // BLK,),
        in_specs=[
            pl.BlockSpec((BLK, W128), lambda i: (i, 0)),
            pl.BlockSpec((BLK, W128), lambda i: (i, 0)),
            pl.BlockSpec((BLK, W128), lambda i: (i, 0)),
            pl.BlockSpec((BLK, W128), lambda i: (i, 0)),
            pl.BlockSpec((BLK, 128), lambda i: (i, 0)),
            pl.BlockSpec((1, D_H), lambda i: (0, 0)),
            pl.BlockSpec((D_H, d_out), lambda i: (0, 0)),
        ],
        out_specs=[pl.BlockSpec((BLK, d_out // n_out), lambda i: (i, 0))
                   for _ in range(n_out)],
        out_shape=[jax.ShapeDtypeStruct((NPAD, d_out // n_out), jnp.float32)
                   for _ in range(n_out)],
    )


_tcmid_a = _make_tcmid(D_H, True)
_tcmid_b = _make_tcmid(D_OUT, False)


def _tcfin_body(agg_ref, hp_ref, dis_ref, b_ref, o_ref):
    dis = dis_ref[:, 0:1]
    u = dis * (agg_ref[...] + hp_ref[...]) + b_ref[...]
    m = jnp.max(u, axis=1, keepdims=True)
    lse = m + jnp.log(jnp.sum(jnp.exp(u - m), axis=1, keepdims=True))
    o_ref[...] = u - lse


_tcfin = pl.pallas_call(
    _tcfin_body,
    grid=(NPAD // BLK,),
    in_specs=[
        pl.BlockSpec((BLK, D_OUT), lambda i: (i, 0)),
        pl.BlockSpec((BLK, D_OUT), lambda i: (i, 0)),
        pl.BlockSpec((BLK, 128), lambda i: (i, 0)),
        pl.BlockSpec((1, D_OUT), lambda i: (0, 0)),
    ],
    out_specs=pl.BlockSpec((BLK, D_OUT), lambda i: (i, 0)),
    out_shape=jax.ShapeDtypeStruct((NPAD, D_OUT), jnp.float32),
)


def kernel(x, edge_index, W1, b1, W2, b2, W3, b3):
    ei = edge_index.astype(jnp.int32)
    src, dst = ei[0], ei[1]
    xp = jnp.zeros((NPAD, D_IN), jnp.float32).at[:N].set(x)

    deg, slists, dlists, cnts = _route(src, dst)
    dis, hp1a, hp1b = _tc0(deg, xp, W1)
    agg1a, agg1b = _agg2(hp1a, hp1b, slists, dlists, cnts)
    hp2a, hp2b = _tcmid_a(agg1a, agg1b, hp1a, hp1b, dis,
                          b1.reshape(1, -1), W2)
    agg2a, agg2b = _agg2(hp2a, hp2b, slists, dlists, cnts)
    hp3 = _tcmid_b(agg2a, agg2b, hp2a, hp2b, dis, b2.reshape(1, -1), W3)[0]
    agg3 = _agg1(hp3, slists, dlists, cnts)[0]
    out = _tcfin(agg3, hp3, dis, b3.reshape(1, -1))
    return out[:N]


# speedup vs baseline: 3.8934x; 1.0315x over previous
import functools

import jax
import jax.numpy as jnp
from jax import lax
from jax.experimental import pallas as pl
from jax.experimental.pallas import tpu as pltpu
from jax.experimental.pallas import tpu_sc as plsc

N = 10000
E = 160000
D_IN = 256
D_H = 256
D_OUT = 128
W128 = 128

NPAD = 10240
NC, NS, LANES = 2, 16, 16
NW = NC * NS
HALF = NPAD // NC
TRASH = HALF
HROWS = HALF + 8
EPT = E // NS
GB = 128
CAP = 10496
NG = CAP // GB
RPT = HALF // NS
BLK = 256


def _mesh():
    return plsc.VectorSubcoreMesh(
        core_axis_name="c", subcore_axis_name="s",
        num_cores=NC, num_subcores=NS)


def _route_body(src_hbm, dst_hbm,
                deg_hbm, sl_hbm, dl_hbm, cnt_hbm,
                sbuf, dbuf, slist, dlist, dlist2, ones_b, zbuf, cntv,
                degacc):
    c = lax.axis_index("c")
    s = lax.axis_index("s")
    wid = s * NC + c
    lo = c * HALF

    pltpu.sync_copy(src_hbm.at[pl.ds(s * EPT, EPT)], sbuf)
    pltpu.sync_copy(dst_hbm.at[pl.ds(s * EPT, EPT)], dbuf)

    zrow = jnp.zeros((LANES,), jnp.float32)
    for i in range(LANES):
        for jj in range(W128 // LANES):
            zbuf[i, pl.ds(jj * LANES, LANES)] = zrow
    for j in range(RPT // LANES):
        pltpu.sync_copy(zbuf, degacc.at[pl.ds(s * RPT + j * LANES, LANES)])

    e1 = (lax.iota(jnp.int32, LANES) == 0).astype(jnp.float32)

    def fill_ones(i, _):
        ones_b[i, pl.ds(0, LANES)] = e1
        for jj in range(1, W128 // LANES):
            ones_b[i, pl.ds(jj * LANES, LANES)] = zrow
        return 0
    lax.fori_loop(0, GB, fill_ones, 0, unroll=False)

    plsc.subcore_barrier()

    def filt(i, cnt):
        d = dbuf[pl.ds(i * LANES, LANES)]
        sv = sbuf[pl.ds(i * LANES, LANES)]
        m = (d >= lo) & (d < lo + HALF)
        plsc.store_compressed(slist.at[pl.ds(cnt, LANES)], sv, mask=m)
        plsc.store_compressed(dlist.at[pl.ds(cnt, LANES)], d - lo, mask=m)
        return cnt + jnp.sum(m.astype(jnp.int32))
    cnt = lax.fori_loop(0, EPT // LANES, filt, 0, unroll=False)

    zero16 = jnp.zeros((LANES,), jnp.int32)
    trash16 = jnp.full((LANES,), TRASH, jnp.int32)
    for t in range(2 * GB // LANES):
        slist[pl.ds(cnt + t * LANES, LANES)] = zero16
        dlist[pl.ds(cnt + t * LANES, LANES)] = trash16
    kpad = ((cnt + GB - 1) // GB) * GB
    nb = kpad // GB

    def relayout(r, _):
        for k in range(GB // LANES):
            dlist2[r, pl.ds(k * LANES, LANES)] = dlist[
                pl.ds(r * GB + k * LANES, LANES)]
        return 0
    lax.fori_loop(0, NG, relayout, 0, unroll=False)

    def dscat(j, _):
        pltpu.sync_copy(ones_b, degacc.at[dlist2.at[j]], add=True)
        return 0
    lax.fori_loop(0, nb, dscat, 0, unroll=False)

    plsc.subcore_barrier()

    pltpu.sync_copy(degacc.at[pl.ds(s * RPT, RPT)],
                    deg_hbm.at[pl.ds(c * HALF + s * RPT, RPT)])
    pltpu.sync_copy(slist, sl_hbm.at[wid])
    pltpu.sync_copy(dlist2, dl_hbm.at[wid])
    cntv[:] = jnp.full((LANES,), kpad, jnp.int32)
    pltpu.sync_copy(cntv, cnt_hbm.at[wid])


_route = pl.kernel(
    _route_body,
    out_type=[
        jax.ShapeDtypeStruct((NPAD, W128), jnp.float32),
        jax.ShapeDtypeStruct((NW, CAP), jnp.int32),
        jax.ShapeDtypeStruct((NW, NG, GB), jnp.int32),
        jax.ShapeDtypeStruct((NW, LANES), jnp.int32),
    ],
    mesh=_mesh(),
    compiler_params=pltpu.CompilerParams(needs_layout_passes=False),
    scratch_types=[
        pltpu.VMEM((EPT,), jnp.int32),
        pltpu.VMEM((EPT,), jnp.int32),
        pltpu.VMEM((CAP,), jnp.int32),
        pltpu.VMEM((CAP,), jnp.int32),
        pltpu.VMEM((NG, GB), jnp.int32),
        pltpu.VMEM((GB, W128), jnp.float32),
        pltpu.VMEM((LANES, W128), jnp.float32),
        pltpu.VMEM((LANES,), jnp.int32),
        pltpu.VMEM_SHARED((HROWS, W128), jnp.float32),
    ],
)


def _agg_body(nplanes, *refs):
    hps = refs[:nplanes]
    sl_hbm, dl_hbm, cnt_hbm = refs[nplanes:nplanes + 3]
    aggs = refs[nplanes + 3:2 * nplanes + 3]
    sbufl, dbufl, cntv, zbuf = refs[2 * nplanes + 3:2 * nplanes + 7]
    stag0, stag1 = refs[2 * nplanes + 7:2 * nplanes + 9]
    sem_g0, sem_g1, sem_s0, sem_l = refs[2 * nplanes + 9:2 * nplanes + 13]
    accs = refs[2 * nplanes + 13:]

    c = lax.axis_index("c")
    s = lax.axis_index("s")
    wid = s * NC + c

    pltpu.sync_copy(cnt_hbm.at[wid], cntv)

    zrow = jnp.zeros((LANES,), jnp.float32)
    for i in range(LANES):
        for jj in range(W128 // LANES):
            zbuf[i, pl.ds(jj * LANES, LANES)] = zrow
    for j in range(RPT // LANES):
        for p in range(nplanes):
            pltpu.sync_copy(zbuf,
                            accs[p].at[pl.ds(s * RPT + j * LANES, LANES)])

    plsc.subcore_barrier()

    nb = cntv[:][0] // GB

    sl_w = sl_hbm.at[wid]
    dl_w = dl_hbm.at[wid]

    def load_lists(j, slot):
        pltpu.sync_copy(sl_w.at[pl.ds(j * GB, GB)], sbufl.at[slot])
        pltpu.sync_copy(dl_w.at[pl.ds(j, 1)], dbufl.at[pl.ds(slot, 1)])

    def load_lists_start(j, slot):
        pltpu.async_copy(sl_w.at[pl.ds(j * GB, GB)], sbufl.at[slot], sem_l)
        pltpu.async_copy(dl_w.at[pl.ds(j, 1)], dbufl.at[pl.ds(slot, 1)],
                         sem_l)

    def load_lists_wait(j, slot):
        pltpu.make_async_copy(sl_w.at[pl.ds(j * GB, GB)], sbufl.at[slot],
                              sem_l).wait()
        pltpu.make_async_copy(dl_w.at[pl.ds(j, 1)],
                              dbufl.at[pl.ds(slot, 1)], sem_l).wait()

    if nplanes == 2:
        @pl.when(nb > 0)
        def _prologue():
            load_lists(0, 0)
            pltpu.async_copy(hps[0].at[sbufl.at[0]], stag0, sem_g0)

        def body(j, _):
            a = j % 2
            b = 1 - a
            load_lists_start(j + 1, b)
            pltpu.make_async_copy(hps[0].at[sbufl.at[a]], stag0,
                                  sem_g0).wait()
            pltpu.async_copy(hps[1].at[sbufl.at[a]], stag1, sem_g1)
            pltpu.async_copy(stag0, accs[0].at[dbufl.at[a]], sem_s0,
                             add=True)
            pltpu.make_async_copy(stag0, accs[0].at[dbufl.at[a]],
                                  sem_s0).wait()
            load_lists_wait(j + 1, b)

            @pl.when(j + 1 < nb)
            def _next():
                pltpu.async_copy(hps[0].at[sbufl.at[b]], stag0, sem_g0)
            pltpu.make_async_copy(hps[1].at[sbufl.at[a]], stag1,
                                  sem_g1).wait()
            pltpu.sync_copy(stag1, accs[1].at[dbufl.at[a]], add=True)
            return 0
        lax.fori_loop(0, nb, body, 0, unroll=False)
    else:
        npairs = nb // 2
        rem = nb - 2 * npairs

        @pl.when(nb > 0)
        def _prologue():
            load_lists(0, 0)
            pltpu.async_copy(hps[0].at[sbufl.at[0]], stag0, sem_g0)

        def pbody(u, _):
            j0 = 2 * u
            load_lists_start(j0 + 1, 1)
            load_lists_wait(j0 + 1, 1)
            pltpu.async_copy(hps[0].at[sbufl.at[1]], stag1, sem_g1)
            pltpu.make_async_copy(hps[0].at[sbufl.at[0]], stag0,
                                  sem_g0).wait()
            pltpu.async_copy(stag0, accs[0].at[dbufl.at[0]], sem_s0,
                             add=True)
            pltpu.make_async_copy(hps[0].at[sbufl.at[1]], stag1,
                                  sem_g1).wait()
            pltpu.sync_copy(stag1, accs[0].at[dbufl.at[1]], add=True)
            pltpu.make_async_copy(stag0, accs[0].at[dbufl.at[0]],
                                  sem_s0).wait()
            load_lists(j0 + 2, 0)
            pltpu.async_copy(hps[0].at[sbufl.at[0]], stag0, sem_g0)
            return 0
        lax.fori_loop(0, npairs, pbody, 0, unroll=False)

        @pl.when(nb > 0)
        def _epilogue():
            pltpu.make_async_copy(hps[0].at[sbufl.at[0]], stag0,
                                  sem_g0).wait()

            @pl.when(rem == 1)
            def _tail():
                pltpu.sync_copy(stag0, accs[0].at[dbufl.at[0]], add=True)

    plsc.subcore_barrier()

    for p in range(nplanes):
        pltpu.sync_copy(accs[p].at[pl.ds(s * RPT, RPT)],
                        aggs[p].at[pl.ds(c * HALF + s * RPT, RPT)])


def _make_agg(nplanes):
    return pl.kernel(
        functools.partial(_agg_body, nplanes),
        out_type=[jax.ShapeDtypeStruct((NPAD, W128), jnp.float32)
                  for _ in range(nplanes)],
        mesh=_mesh(),
        compiler_params=pltpu.CompilerParams(needs_layout_passes=False),
        scratch_types=(
            [pltpu.VMEM((2, GB), jnp.int32),
             pltpu.VMEM((2, GB), jnp.int32),
             pltpu.VMEM((LANES,), jnp.int32),
             pltpu.VMEM((LANES, W128), jnp.float32),
             pltpu.VMEM((GB, W128), jnp.float32),
             pltpu.VMEM((GB, W128), jnp.float32),
             pltpu.SemaphoreType.DMA,
             pltpu.SemaphoreType.DMA,
             pltpu.SemaphoreType.DMA,
             pltpu.SemaphoreType.DMA]
            + [pltpu.VMEM_SHARED((HROWS, W128), jnp.float32)
               for _ in range(nplanes)]
        ),
    )


_agg2 = _make_agg(2)
_agg1 = _make_agg(1)


def _tc0_body(deg_ref, x_ref, w_ref, dis_ref, hp0_ref, hp1_ref):
    dis = lax.rsqrt(deg_ref[:, 0:1] + 1.0)
    dis_ref[...] = jnp.broadcast_to(dis, (BLK, 128))
    h = dis * jnp.dot(x_ref[...], w_ref[...],
                      preferred_element_type=jnp.float32)
    hp0_ref[...] = h[:, :W128]
    hp1_ref[...] = h[:, W128:]


_tc0 = pl.pallas_call(
    _tc0_body,
    grid=(NPAD // BLK,),
    in_specs=[
        pl.BlockSpec((BLK, W128), lambda i: (i, 0)),
        pl.BlockSpec((BLK, D_IN), lambda i: (i, 0)),
        pl.BlockSpec((D_IN, D_H), lambda i: (0, 0)),
    ],
    out_specs=[
        pl.BlockSpec((BLK, 128), lambda i: (i, 0)),
        pl.BlockSpec((BLK, W128), lambda i: (i, 0)),
        pl.BlockSpec((BLK, W128), lambda i: (i, 0)),
    ],
    out_shape=[
        jax.ShapeDtypeStruct((NPAD, 128), jnp.float32),
        jax.ShapeDtypeStruct((NPAD, W128), jnp.float32),
        jax.ShapeDtypeStruct((NPAD, W128), jnp.float32),
    ],
)


def _tcmid_body(split_out, a0_ref, a1_ref, h0_ref, h1_ref, dis_ref, b_ref,
                w_ref, *o_refs):
    dis = dis_ref[:, 0:1]
    agg = jnp.concatenate([a0_ref[...], a1_ref[...]], axis=1)
    hp = jnp.concatenate([h0_ref[...], h1_ref[...]], axis=1)
    t = jnp.maximum(dis * (agg + hp) + b_ref[...], 0.0)
    out = dis * jnp.dot(t, w_ref[...], preferred_element_type=jnp.float32)
    if split_out:
        o_refs[0][...] = out[:, :W128]
        o_refs[1][...] = out[:, W128:]
    else:
        o_refs[0][...] = out


def _make_tcmid(d_out, split_out):
    n_out = 2 if split_out else 1
    return pl.pallas_call(
        functools.partial(_tcmid_body, split_out),
        grid=(NPAD // BLK,),
        in_specs=[
            pl.BlockSpec((BLK, W128), lambda i: (i, 0)),
            pl.BlockSpec((BLK, W128), lambda i: (i, 0)),
            pl.BlockSpec((BLK, W128), lambda i: (i, 0)),
            pl.BlockSpec((BLK, W128), lambda i: (i, 0)),
            pl.BlockSpec((BLK, 128), lambda i: (i, 0)),
            pl.BlockSpec((1, D_H), lambda i: (0, 0)),
            pl.BlockSpec((D_H, d_out), lambda i: (0, 0)),
        ],
        out_specs=[pl.BlockSpec((BLK, d_out // n_out), lambda i: (i, 0))
                   for _ in range(n_out)],
        out_shape=[jax.ShapeDtypeStruct((NPAD, d_out // n_out), jnp.float32)
                   for _ in range(n_out)],
    )


_tcmid_a = _make_tcmid(D_H, True)
_tcmid_b = _make_tcmid(D_OUT, False)


def _tcfin_body(agg_ref, hp_ref, dis_ref, b_ref, o_ref):
    dis = dis_ref[:, 0:1]
    u = dis * (agg_ref[...] + hp_ref[...]) + b_ref[...]
    m = jnp.max(u, axis=1, keepdims=True)
    lse = m + jnp.log(jnp.sum(jnp.exp(u - m), axis=1, keepdims=True))
    o_ref[...] = u - lse


_tcfin = pl.pallas_call(
    _tcfin_body,
    grid=(NPAD // BLK,),
    in_specs=[
        pl.BlockSpec((BLK, D_OUT), lambda i: (i, 0)),
        pl.BlockSpec((BLK, D_OUT), lambda i: (i, 0)),
        pl.BlockSpec((BLK, 128), lambda i: (i, 0)),
        pl.BlockSpec((1, D_OUT), lambda i: (0, 0)),
    ],
    out_specs=pl.BlockSpec((BLK, D_OUT), lambda i: (i, 0)),
    out_shape=jax.ShapeDtypeStruct((NPAD, D_OUT), jnp.float32),
)


def kernel(x, edge_index, W1, b1, W2, b2, W3, b3):
    ei = edge_index.astype(jnp.int32)
    src, dst = ei[0], ei[1]
    xp = jnp.zeros((NPAD, D_IN), jnp.float32).at[:N].set(x)

    deg, slists, dlists, cnts = _route(src, dst)
    dis, hp1a, hp1b = _tc0(deg, xp, W1)
    agg1a, agg1b = _agg2(hp1a, hp1b, slists, dlists, cnts)
    hp2a, hp2b = _tcmid_a(agg1a, agg1b, hp1a, hp1b, dis,
                          b1.reshape(1, -1), W2)
    agg2a, agg2b = _agg2(hp2a, hp2b, slists, dlists, cnts)
    hp3 = _tcmid_b(agg2a, agg2b, hp2a, hp2b, dis, b2.reshape(1, -1), W3)[0]
    agg3 = _agg1(hp3, slists, dlists, cnts)[0]
    out = _tcfin(agg3, hp3, dis, b3.reshape(1, -1))
    return out[:N]
